# baseline (speedup 1.0000x reference)
import jax
import jax.numpy as jnp
from jax import lax
from jax.experimental import pallas as pl
from jax.experimental.pallas import tpu as pltpu
from jax.experimental.pallas import tpu_sc as plsc

B = 16384
F = 26
NC = 2
NS = 16
NW = NC * NS
BPW = B // NW
IPW = BPW * F
CHUNK = 128
NJ = IPW // CHUNK
GRP = 8


def _body(x_hbm, tab_hbm, bias_hbm, out_hbm, idx_v, buf_v, acc_v, bias_v, sem):
    cid = lax.axis_index("c")
    sid = lax.axis_index("s")
    wid = sid * NC + cid

    pltpu.sync_copy(x_hbm.at[wid], idx_v)
    pltpu.sync_copy(bias_hbm, bias_v)
    binit = bias_v[...]

    def fire(g):
        cps = []
        for jj in range(GRP):
            j = g * GRP + jj
            cps.append(
                pltpu.async_copy(
                    tab_hbm.at[idx_v.at[pl.ds(j * CHUNK, CHUNK)]],
                    buf_v.at[pl.ds(j * CHUNK, CHUNK)],
                    sem,
                )
            )
        return cps

    prev = None
    for g in range(NJ // GRP):
        cur = fire(g)
        if prev is not None:
            for cp in prev:
                cp.wait()
        prev = cur
    for cp in prev:
        cp.wait()

    iotaF = lax.iota(jnp.int32, 16) * F
    for g in range(BPW // 16):
        acc16 = binit
        base = g * 16 * F
        for f in range(F):
            acc16 = acc16 + plsc.load_gather(buf_v, [iotaF + (base + f)])
        acc_v[pl.ds(g * 16, 16)] = acc16

    pltpu.sync_copy(acc_v, out_hbm.at[pl.ds(wid * BPW, BPW)])


@jax.jit
def _linear_features(xw, tab, bias):
    mesh = plsc.VectorSubcoreMesh(core_axis_name="c", subcore_axis_name="s")
    return pl.kernel(
        _body,
        out_type=jax.ShapeDtypeStruct((B,), jnp.float32),
        mesh=mesh,
        compiler_params=pltpu.CompilerParams(needs_layout_passes=False),
        scratch_types=[
            pltpu.VMEM((IPW,), jnp.int32),
            pltpu.VMEM((IPW,), jnp.float32),
            pltpu.VMEM((BPW,), jnp.float32),
            pltpu.VMEM((16,), jnp.float32),
            pltpu.SemaphoreType.DMA,
        ],
    )(xw, tab, bias)


def kernel(x, fc_weight, bias):
    xw = x.astype(jnp.int32).reshape(NW, IPW)
    out = _linear_features(
        xw, fc_weight.reshape(-1), jnp.broadcast_to(bias, (16,))
    )
    return out.reshape(B, 1)
